# Initial kernel scaffold; baseline (speedup 1.0000x reference)
import jax
import jax.numpy as jnp
from jax.experimental import pallas as pl


def kernel(z, embeddings):
    raise NotImplementedError("write your pallas kernel here")


# speedup vs baseline: 1.4536x; 1.4536x over previous
import functools

import jax
import jax.numpy as jnp
from jax.experimental import pallas as pl

_TOK = 2048


def _vq_body(z_ref, e_ref, zq_ref, idx_ref):
    z = z_ref[...]
    e = e_ref[...]
    zn = jnp.sum(z * z, axis=1, keepdims=True)
    en = jnp.sum(e * e, axis=1)[None, :]
    cross = jax.lax.dot_general(z, e, (((1,), (1,)), ((), ())))
    d = zn + en - 2.0 * cross
    dmin = jnp.min(d, axis=1, keepdims=True)
    n_iota = jax.lax.broadcasted_iota(jnp.int32, d.shape, 1)
    idx = jnp.min(jnp.where(d == dmin, n_iota, jnp.int32(2**30)), axis=1)
    idx_ref[0, 0, :] = idx
    onehot = (n_iota == idx[:, None]).astype(jnp.float32)
    zq_ref[...] = jax.lax.dot_general(onehot, e, (((1,), (0,)), ((), ())))


def kernel(z, embeddings):
    e_dim = z.shape[-1]
    zf = z.reshape(-1, e_dim)
    n_tok = zf.shape[0]
    n_codes = embeddings.shape[0]
    grid = n_tok // _TOK
    zq, idx = pl.pallas_call(
        _vq_body,
        grid=(grid,),
        in_specs=[
            pl.BlockSpec((_TOK, e_dim), lambda i: (i, 0)),
            pl.BlockSpec((n_codes, e_dim), lambda i: (0, 0)),
        ],
        out_specs=[
            pl.BlockSpec((_TOK, e_dim), lambda i: (i, 0)),
            pl.BlockSpec((1, 1, _TOK), lambda i: (i, 0, 0)),
        ],
        out_shape=[
            jax.ShapeDtypeStruct((n_tok, e_dim), jnp.float32),
            jax.ShapeDtypeStruct((grid, 1, _TOK), jnp.int32),
        ],
    )(zf, embeddings)
    return zq.reshape(z.shape), idx.reshape(z.shape[:-1])


# speedup vs baseline: 1.9162x; 1.3183x over previous
import functools

import jax
import jax.numpy as jnp
from jax.experimental import pallas as pl

_TOK = 2048


def _vq_body(z_ref, em2_ref, en_ref, e_ref, zq_ref, idx_ref):
    zb = z_ref[...]
    em2 = em2_ref[...]
    en = en_ref[...]
    s = jax.lax.dot_general(em2, zb, (((1,), (1,)), ((), ()))) + en
    m = jnp.min(s, axis=0)
    n_iota = jax.lax.broadcasted_iota(jnp.int32, s.shape, 0)
    idx = jnp.min(jnp.where(s == m[None, :], n_iota, jnp.int32(2**30)),
                  axis=0)
    idx_ref[0, 0, :] = idx
    onehot = (n_iota == idx[None, :]).astype(jnp.float32)
    zq_ref[...] = jax.lax.dot_general(onehot, e_ref[...],
                                      (((0,), (0,)), ((), ())))


def kernel(z, embeddings):
    e_dim = z.shape[-1]
    zf = z.reshape(-1, e_dim)
    n_tok = zf.shape[0]
    n_codes = embeddings.shape[0]
    grid = n_tok // _TOK
    em2 = embeddings * -2.0
    en = jnp.sum(embeddings * embeddings, axis=1, keepdims=True)
    zq, idx = pl.pallas_call(
        _vq_body,
        grid=(grid,),
        in_specs=[
            pl.BlockSpec((_TOK, e_dim), lambda i: (i, 0)),
            pl.BlockSpec((n_codes, e_dim), lambda i: (0, 0)),
            pl.BlockSpec((n_codes, 1), lambda i: (0, 0)),
            pl.BlockSpec((n_codes, e_dim), lambda i: (0, 0)),
        ],
        out_specs=[
            pl.BlockSpec((_TOK, e_dim), lambda i: (i, 0)),
            pl.BlockSpec((1, 1, _TOK), lambda i: (i, 0, 0)),
        ],
        out_shape=[
            jax.ShapeDtypeStruct((n_tok, e_dim), jnp.float32),
            jax.ShapeDtypeStruct((grid, 1, _TOK), jnp.int32),
        ],
    )(zf, em2, en, embeddings)
    return zq.reshape(z.shape), idx.reshape(z.shape[:-1])
